# Initial kernel scaffold; baseline (speedup 1.0000x reference)
import jax
import jax.numpy as jnp
from jax.experimental import pallas as pl


def kernel(x, edge_index, W, b):
    raise NotImplementedError("write your pallas kernel here")


# speedup vs baseline: 25.9043x; 25.9043x over previous
import functools

import jax
import jax.numpy as jnp
from jax import lax
from jax.experimental import pallas as pl
from jax.experimental.pallas import tpu as pltpu
from jax.experimental.pallas import tpu_sc as plsc

N = 10000
NP = 10240
E = 320000
D = 128
DH = D // 2
NC, NS = 2, 16
NW = NC * NS
KE = 128
EPW = 10240
E_PAD = EPW * NW
NCH = EPW // KE
NCHH = E_PAD // (NS * KE)
RPT = NP // NS
NZR = 16

f32 = jnp.float32
i32 = jnp.int32

_mesh = plsc.VectorSubcoreMesh(core_axis_name="c", subcore_axis_name="s")


def _fill_1d(ref, n, val):
    v = jnp.full((16,), val, f32)
    for i in range(n // 16):
        ref[pl.ds(i * 16, 16)] = v


@functools.partial(
    pl.kernel,
    mesh=_mesh,
    compiler_params=pltpu.CompilerParams(use_tc_tiling_on_sc=False),
    out_type=jax.ShapeDtypeStruct((NC, NP), f32),
    scratch_types=[
        pltpu.VMEM((NCH, KE), i32),
        pltpu.VMEM((KE,), f32),
        pltpu.VMEM((RPT,), f32),
        pltpu.VMEM_SHARED((NP,), f32),
    ],
)
def _deg_kernel(dst_hbm, out_hbm, dall_v, ones_v, row_v, deg_sp):
    cid = lax.axis_index("c")
    sid = lax.axis_index("s")
    wid = sid * NC + cid
    _fill_1d(ones_v, KE, 1.0)
    _fill_1d(row_v, RPT, 0.0)
    pltpu.sync_copy(row_v, deg_sp.at[pl.ds(sid * RPT, RPT)])
    pltpu.sync_copy(dst_hbm.at[pl.ds(wid * NCH, NCH)], dall_v)
    plsc.subcore_barrier()

    def body(c, carry):
        pltpu.sync_copy(ones_v, deg_sp.at[dall_v.at[c]], add=True)
        return carry

    lax.fori_loop(0, NCH, body, jnp.int32(0))
    plsc.subcore_barrier()
    pltpu.sync_copy(deg_sp.at[pl.ds(sid * RPT, RPT)], row_v)
    pltpu.sync_copy(row_v, out_hbm.at[cid, pl.ds(sid * RPT, RPT)])


@functools.partial(
    pl.kernel,
    mesh=_mesh,
    compiler_params=pltpu.CompilerParams(use_tc_tiling_on_sc=False),
    out_type=jax.ShapeDtypeStruct((NC, NP, DH), f32),
    scratch_types=[
        pltpu.VMEM((NCHH, KE), i32),
        pltpu.VMEM((NCHH, KE), i32),
        pltpu.VMEM((KE, DH), f32),
        pltpu.VMEM((KE, DH), f32),
        pltpu.VMEM((NZR, DH), f32),
        pltpu.SemaphoreType.DMA,
        pltpu.SemaphoreType.DMA,
        pltpu.VMEM_SHARED((NP, DH), f32),
    ],
)
def _hop_kernel(h_hbm, src_hbm, dst_hbm, out_hbm,
                sall_v, dall_v, r0_v, r1_v, zero_v, sem0, sem1, acc_sp):
    cid = lax.axis_index("c")
    sid = lax.axis_index("s")
    rbuf = (r0_v, r1_v)
    sems = (sem0, sem1)

    zvec = jnp.zeros((16,), f32)
    for i in range(NZR):
        for j in range(DH // 16):
            zero_v[i, pl.ds(j * 16, 16)] = zvec
    for k in range(RPT // NZR):
        pltpu.sync_copy(zero_v, acc_sp.at[pl.ds(sid * RPT + k * NZR, NZR)])

    pltpu.sync_copy(src_hbm.at[pl.ds(sid * NCHH, NCHH)], sall_v)
    pltpu.sync_copy(dst_hbm.at[pl.ds(sid * NCHH, NCHH)], dall_v)
    plsc.subcore_barrier()

    for b in range(2):
        pltpu.async_copy(h_hbm.at[cid].at[sall_v.at[b]], rbuf[b], sems[b])

    def pair(g, carry):
        for b in range(2):
            c = g * 2 + b
            pltpu.make_async_copy(
                h_hbm.at[cid].at[sall_v.at[c]], rbuf[b], sems[b]).wait()
            pltpu.sync_copy(rbuf[b], acc_sp.at[dall_v.at[c]], add=True)
            c2 = c + 2

            @pl.when(c2 < NCHH)
            def _():
                pltpu.async_copy(
                    h_hbm.at[cid].at[sall_v.at[c2]], rbuf[b], sems[b])

        return carry

    lax.fori_loop(0, NCHH // 2, pair, jnp.int32(0))
    plsc.subcore_barrier()
    pltpu.sync_copy(acc_sp.at[pl.ds(sid * RPT, RPT)],
                    out_hbm.at[cid, pl.ds(sid * RPT, RPT)])


def _split(res):
    return jnp.stack([res[:, :DH], res[:, DH:]])


def _prep_body(x_ref, d0_ref, d1_ref, xts_ref, xtf_ref, dis_ref, dinv_ref):
    deg = d0_ref[...] + d1_ref[...] + 1.0
    dis = lax.rsqrt(deg)
    dis_ref[...] = dis
    dinv_ref[...] = 1.0 / deg
    xt = x_ref[...] * dis
    xtf_ref[...] = xt
    xts_ref[...] = _split(xt)


_prep = pl.pallas_call(
    _prep_body,
    grid=(NP // 1024,),
    in_specs=[
        pl.BlockSpec((1024, D), lambda i: (i, 0)),
        pl.BlockSpec((1024, 1), lambda i: (i, 0)),
        pl.BlockSpec((1024, 1), lambda i: (i, 0)),
    ],
    out_specs=[
        pl.BlockSpec((NC, 1024, DH), lambda i: (0, i, 0)),
        pl.BlockSpec((1024, D), lambda i: (i, 0)),
        pl.BlockSpec((1024, 1), lambda i: (i, 0)),
        pl.BlockSpec((1024, 1), lambda i: (i, 0)),
    ],
    out_shape=[
        jax.ShapeDtypeStruct((NC, NP, DH), f32),
        jax.ShapeDtypeStruct((NP, D), f32),
        jax.ShapeDtypeStruct((NP, 1), f32),
        jax.ShapeDtypeStruct((NP, 1), f32),
    ],
)


def _combine_body(p_ref, base_ref, sc_ref, hs_ref, hf_ref):
    ph = jnp.concatenate([p_ref[0], p_ref[1]], axis=1)
    res = (ph + base_ref[...]) * sc_ref[...]
    hf_ref[...] = res
    hs_ref[...] = _split(res)


_combine = pl.pallas_call(
    _combine_body,
    grid=(NP // 1024,),
    in_specs=[
        pl.BlockSpec((NC, 1024, DH), lambda i: (0, i, 0)),
        pl.BlockSpec((1024, D), lambda i: (i, 0)),
        pl.BlockSpec((1024, 1), lambda i: (i, 0)),
    ],
    out_specs=[
        pl.BlockSpec((NC, 1024, DH), lambda i: (0, i, 0)),
        pl.BlockSpec((1024, D), lambda i: (i, 0)),
    ],
    out_shape=[
        jax.ShapeDtypeStruct((NC, NP, DH), f32),
        jax.ShapeDtypeStruct((NP, D), f32),
    ],
)


def _final_body(q_ref, base_ref, sc_ref, w_ref, b_ref, out_ref):
    qh = jnp.concatenate([q_ref[0], q_ref[1]], axis=1)
    h = (qh + base_ref[...]) * sc_ref[...]
    out_ref[...] = lax.dot_general(
        h, w_ref[...], (((1,), (1,)), ((), ())),
        preferred_element_type=f32) + b_ref[...]


_final = pl.pallas_call(
    _final_body,
    grid=(NP // 1024,),
    in_specs=[
        pl.BlockSpec((NC, 1024, DH), lambda i: (0, i, 0)),
        pl.BlockSpec((1024, D), lambda i: (i, 0)),
        pl.BlockSpec((1024, 1), lambda i: (i, 0)),
        pl.BlockSpec((D, D), lambda i: (0, 0)),
        pl.BlockSpec((1, D), lambda i: (0, 0)),
    ],
    out_specs=pl.BlockSpec((1024, D), lambda i: (i, 0)),
    out_shape=jax.ShapeDtypeStruct((NP, D), f32),
)


def kernel(x, edge_index, W, b):
    ei = edge_index.astype(i32)
    pad = N + (jnp.arange(E_PAD - E, dtype=i32) % (NP - N))
    src = jnp.concatenate([ei[0], pad]).reshape(E_PAD // KE, KE)
    dst = jnp.concatenate([ei[1], pad]).reshape(E_PAD // KE, KE)
    x_p = jnp.pad(x, ((0, NP - N), (0, 0)))

    degp = _deg_kernel(dst)
    xts, xtf, dis, dinv = _prep(
        x_p, degp[0].reshape(NP, 1), degp[1].reshape(NP, 1))
    p = _hop_kernel(xts, src, dst)
    h1s, h1f = _combine(p, xtf, dinv)
    q = _hop_kernel(h1s, src, dst)
    out = _final(q, h1f, dis, W, b.reshape(1, D))
    return out[:N]


# speedup vs baseline: 28.4906x; 1.0998x over previous
import functools

import jax
import jax.numpy as jnp
from jax import lax
from jax.experimental import pallas as pl
from jax.experimental.pallas import tpu as pltpu
from jax.experimental.pallas import tpu_sc as plsc

N = 10000
NP = 10240
E = 320000
D = 128
DH = D // 2
NC, NS = 2, 16
NW = NC * NS
KE = 512
EPW = 10240
E_PAD = EPW * NW
NCH = EPW // KE
NCHH = E_PAD // (NS * KE)
RPT = NP // NS
NZR = 16

f32 = jnp.float32
i32 = jnp.int32

_mesh = plsc.VectorSubcoreMesh(core_axis_name="c", subcore_axis_name="s")


def _fill_1d(ref, n, val):
    v = jnp.full((16,), val, f32)
    for i in range(n // 16):
        ref[pl.ds(i * 16, 16)] = v


@functools.partial(
    pl.kernel,
    mesh=_mesh,
    compiler_params=pltpu.CompilerParams(use_tc_tiling_on_sc=False),
    out_type=jax.ShapeDtypeStruct((NC, NP), f32),
    scratch_types=[
        pltpu.VMEM((NCH, KE), i32),
        pltpu.VMEM((KE,), f32),
        pltpu.VMEM((RPT,), f32),
        pltpu.VMEM_SHARED((NP,), f32),
    ],
)
def _deg_kernel(dst_hbm, out_hbm, dall_v, ones_v, row_v, deg_sp):
    cid = lax.axis_index("c")
    sid = lax.axis_index("s")
    wid = sid * NC + cid
    _fill_1d(ones_v, KE, 1.0)
    _fill_1d(row_v, RPT, 0.0)
    pltpu.sync_copy(row_v, deg_sp.at[pl.ds(sid * RPT, RPT)])
    pltpu.sync_copy(dst_hbm.at[pl.ds(wid * NCH, NCH)], dall_v)
    plsc.subcore_barrier()

    def body(c, carry):
        pltpu.sync_copy(ones_v, deg_sp.at[dall_v.at[c]], add=True)
        return carry

    lax.fori_loop(0, NCH, body, jnp.int32(0))
    plsc.subcore_barrier()
    pltpu.sync_copy(deg_sp.at[pl.ds(sid * RPT, RPT)], row_v)
    pltpu.sync_copy(row_v, out_hbm.at[cid, pl.ds(sid * RPT, RPT)])


@functools.partial(
    pl.kernel,
    mesh=_mesh,
    compiler_params=pltpu.CompilerParams(use_tc_tiling_on_sc=False),
    out_type=jax.ShapeDtypeStruct((NC, NP, DH), f32),
    scratch_types=[
        [pltpu.VMEM((KE,), i32) for _ in range(4)],
        [pltpu.VMEM((KE,), i32) for _ in range(4)],
        [pltpu.VMEM((KE, DH), f32) for _ in range(2)],
        pltpu.VMEM((NZR, DH), f32),
        [pltpu.SemaphoreType.DMA for _ in range(4)],
        [pltpu.SemaphoreType.DMA for _ in range(2)],
        pltpu.VMEM_SHARED((NP, DH), f32),
    ],
)
def _hop_kernel(h_hbm, src_hbm, dst_hbm, out_hbm,
                sidx, didx, rbuf, zero_v, isems, gsems, acc_sp):
    cid = lax.axis_index("c")
    sid = lax.axis_index("s")

    def idx_load(c, q):
        row = sid * NCHH + c
        pltpu.async_copy(src_hbm.at[row], sidx[q], isems[q])
        pltpu.async_copy(dst_hbm.at[row], didx[q], isems[q])

    def idx_wait(c, q):
        row = sid * NCHH + c
        pltpu.make_async_copy(src_hbm.at[row], sidx[q], isems[q]).wait()
        pltpu.make_async_copy(dst_hbm.at[row], didx[q], isems[q]).wait()

    def gather_start(q, b):
        pltpu.async_copy(h_hbm.at[cid].at[sidx[q]], rbuf[b], gsems[b])

    def gather_wait(q, b):
        pltpu.make_async_copy(
            h_hbm.at[cid].at[sidx[q]], rbuf[b], gsems[b]).wait()

    zvec = jnp.zeros((16,), f32)
    for i in range(NZR):
        for j in range(DH // 16):
            zero_v[i, pl.ds(j * 16, 16)] = zvec
    for k in range(RPT // NZR):
        pltpu.sync_copy(zero_v, acc_sp.at[pl.ds(sid * RPT + k * NZR, NZR)])
    plsc.subcore_barrier()

    for q in range(4):
        idx_load(q, q)
    idx_wait(0, 0)
    gather_start(0, 0)

    def quad(g, carry):
        for b4 in range(4):
            c = g * 4 + b4
            b = b4 % 2
            gather_wait(b4, b)
            c1 = c + 1

            @pl.when(c1 < NCHH)
            def _():
                idx_wait(c1, (b4 + 1) % 4)
                gather_start((b4 + 1) % 4, 1 - b)

            pltpu.sync_copy(rbuf[b], acc_sp.at[didx[b4]], add=True)
            c4 = c + 4

            @pl.when(c4 < NCHH)
            def _():
                idx_load(c4, b4)

        return carry

    lax.fori_loop(0, NCHH // 4, quad, jnp.int32(0))
    plsc.subcore_barrier()
    pltpu.sync_copy(acc_sp.at[pl.ds(sid * RPT, RPT)],
                    out_hbm.at[cid, pl.ds(sid * RPT, RPT)])


def _split(res):
    return jnp.stack([res[:, :DH], res[:, DH:]])


def _prep_body(x_ref, d0_ref, d1_ref, xts_ref, xtf_ref, dis_ref, dinv_ref):
    deg = d0_ref[...] + d1_ref[...] + 1.0
    dis = lax.rsqrt(deg)
    dis_ref[...] = dis
    dinv_ref[...] = 1.0 / deg
    xt = x_ref[...] * dis
    xtf_ref[...] = xt
    xts_ref[...] = _split(xt)


_prep = pl.pallas_call(
    _prep_body,
    grid=(NP // 1024,),
    in_specs=[
        pl.BlockSpec((1024, D), lambda i: (i, 0)),
        pl.BlockSpec((1024, 1), lambda i: (i, 0)),
        pl.BlockSpec((1024, 1), lambda i: (i, 0)),
    ],
    out_specs=[
        pl.BlockSpec((NC, 1024, DH), lambda i: (0, i, 0)),
        pl.BlockSpec((1024, D), lambda i: (i, 0)),
        pl.BlockSpec((1024, 1), lambda i: (i, 0)),
        pl.BlockSpec((1024, 1), lambda i: (i, 0)),
    ],
    out_shape=[
        jax.ShapeDtypeStruct((NC, NP, DH), f32),
        jax.ShapeDtypeStruct((NP, D), f32),
        jax.ShapeDtypeStruct((NP, 1), f32),
        jax.ShapeDtypeStruct((NP, 1), f32),
    ],
)


def _combine_body(p_ref, base_ref, sc_ref, hs_ref, hf_ref):
    ph = jnp.concatenate([p_ref[0], p_ref[1]], axis=1)
    res = (ph + base_ref[...]) * sc_ref[...]
    hf_ref[...] = res
    hs_ref[...] = _split(res)


_combine = pl.pallas_call(
    _combine_body,
    grid=(NP // 1024,),
    in_specs=[
        pl.BlockSpec((NC, 1024, DH), lambda i: (0, i, 0)),
        pl.BlockSpec((1024, D), lambda i: (i, 0)),
        pl.BlockSpec((1024, 1), lambda i: (i, 0)),
    ],
    out_specs=[
        pl.BlockSpec((NC, 1024, DH), lambda i: (0, i, 0)),
        pl.BlockSpec((1024, D), lambda i: (i, 0)),
    ],
    out_shape=[
        jax.ShapeDtypeStruct((NC, NP, DH), f32),
        jax.ShapeDtypeStruct((NP, D), f32),
    ],
)


def _final_body(q_ref, base_ref, sc_ref, w_ref, b_ref, out_ref):
    qh = jnp.concatenate([q_ref[0], q_ref[1]], axis=1)
    h = (qh + base_ref[...]) * sc_ref[...]
    out_ref[...] = lax.dot_general(
        h, w_ref[...], (((1,), (1,)), ((), ())),
        preferred_element_type=f32) + b_ref[...]


_final = pl.pallas_call(
    _final_body,
    grid=(NP // 1024,),
    in_specs=[
        pl.BlockSpec((NC, 1024, DH), lambda i: (0, i, 0)),
        pl.BlockSpec((1024, D), lambda i: (i, 0)),
        pl.BlockSpec((1024, 1), lambda i: (i, 0)),
        pl.BlockSpec((D, D), lambda i: (0, 0)),
        pl.BlockSpec((1, D), lambda i: (0, 0)),
    ],
    out_specs=pl.BlockSpec((1024, D), lambda i: (i, 0)),
    out_shape=jax.ShapeDtypeStruct((NP, D), f32),
)


def kernel(x, edge_index, W, b):
    ei = edge_index.astype(i32)
    pad = N + (jnp.arange(E_PAD - E, dtype=i32) % (NP - N))
    src = jnp.concatenate([ei[0], pad]).reshape(E_PAD // KE, KE)
    dst = jnp.concatenate([ei[1], pad]).reshape(E_PAD // KE, KE)
    x_p = jnp.pad(x, ((0, NP - N), (0, 0)))

    degp = _deg_kernel(dst)
    xts, xtf, dis, dinv = _prep(
        x_p, degp[0].reshape(NP, 1), degp[1].reshape(NP, 1))
    p = _hop_kernel(xts, src, dst)
    h1s, h1f = _combine(p, xtf, dinv)
    q = _hop_kernel(h1s, src, dst)
    out = _final(q, h1f, dis, W, b.reshape(1, D))
    return out[:N]


# speedup vs baseline: 30.4176x; 1.0676x over previous
import functools

import jax
import jax.numpy as jnp
from jax import lax
from jax.experimental import pallas as pl
from jax.experimental.pallas import tpu as pltpu
from jax.experimental.pallas import tpu_sc as plsc

N = 10000
NP = 10240
E = 320000
D = 128
DH = D // 2
NC, NS = 2, 16
KE = 512
E_PAD = 327680
NCHH = E_PAD // (NS * KE)
RPT = NP // NS
HR = RPT // 2
NZR = 16

f32 = jnp.float32
i32 = jnp.int32

_mesh = plsc.VectorSubcoreMesh(core_axis_name="c", subcore_axis_name="s")
_sc_params = pltpu.CompilerParams(use_tc_tiling_on_sc=False)


def _rsqrt_newton(d):
    y = lax.bitcast_convert_type(
        jnp.int32(0x5F3759DF) - (lax.bitcast_convert_type(d, i32) >> 1), f32)
    for _ in range(3):
        y = y * (1.5 - 0.5 * d * y * y)
    return y


@functools.partial(
    pl.kernel,
    mesh=_mesh,
    compiler_params=_sc_params,
    out_type=(
        jax.ShapeDtypeStruct((NC, NP, DH), f32),
        jax.ShapeDtypeStruct((NP,), f32),
        jax.ShapeDtypeStruct((NP,), f32),
    ),
    scratch_types=[
        [pltpu.VMEM((KE,), i32) for _ in range(4)],
        pltpu.VMEM((KE,), f32),
        pltpu.VMEM((RPT,), f32),
        pltpu.VMEM((RPT,), f32),
        pltpu.VMEM((RPT,), f32),
        pltpu.VMEM((HR, D), f32),
        pltpu.VMEM((HR, DH), f32),
        [pltpu.SemaphoreType.DMA for _ in range(4)],
        pltpu.VMEM_SHARED((NP,), f32),
    ],
)
def _prep_kernel(dst_hbm, x_hbm, xts_hbm, dis_hbm, dinv_hbm,
                 didx, ones_v, deg_v, dis_v, dinv_v, xb_v, ob_v,
                 isems, deg_sp):
    cid = lax.axis_index("c")
    sid = lax.axis_index("s")

    def fill(ref, n, val):
        v = jnp.full((16,), val, f32)
        for i in range(n // 16):
            ref[pl.ds(i * 16, 16)] = v

    def idx_load(c, q):
        pltpu.async_copy(dst_hbm.at[sid * NCHH + c], didx[q], isems[q])

    def idx_wait(c, q):
        pltpu.make_async_copy(
            dst_hbm.at[sid * NCHH + c], didx[q], isems[q]).wait()

    fill(ones_v, KE, 1.0)
    fill(deg_v, RPT, 0.0)
    pltpu.sync_copy(deg_v, deg_sp.at[pl.ds(sid * RPT, RPT)])
    plsc.subcore_barrier()

    for q in range(4):
        idx_load(q, q)

    def quad(g, carry):
        for b4 in range(4):
            c = g * 4 + b4
            idx_wait(c, b4)
            pltpu.sync_copy(ones_v, deg_sp.at[didx[b4]], add=True)
            c4 = c + 4

            @pl.when(c4 < NCHH)
            def _():
                idx_load(c4, b4)

        return carry

    lax.fori_loop(0, NCHH // 4, quad, jnp.int32(0))
    plsc.subcore_barrier()

    base = sid * RPT
    pltpu.sync_copy(deg_sp.at[pl.ds(base, RPT)], deg_v)

    def grp(i, carry):
        d = deg_v[pl.ds(i * 16, 16)] + 1.0
        y = _rsqrt_newton(d)
        dis_v[pl.ds(i * 16, 16)] = y
        dinv_v[pl.ds(i * 16, 16)] = 1.0 / d
        return carry

    lax.fori_loop(0, RPT // 16, grp, jnp.int32(0))

    @pl.when(cid == 0)
    def _():
        pltpu.sync_copy(dis_v, dis_hbm.at[pl.ds(base, RPT)])
        pltpu.sync_copy(dinv_v, dinv_hbm.at[pl.ds(base, RPT)])

    for k in range(2):
        rb = base + k * HR
        pltpu.sync_copy(x_hbm.at[pl.ds(rb, HR)], xb_v)

        def rgrp(g, carry):
            sv = dis_v[pl.ds(k * HR + g * 16, 16)]
            for rr in range(16):
                r = g * 16 + rr
                s = jnp.full((16,), sv[rr], f32)
                for j in range(DH // 16):
                    ob_v[r, pl.ds(j * 16, 16)] = (
                        xb_v[r, pl.ds(cid * DH + j * 16, 16)] * s)
            return carry

        lax.fori_loop(0, HR // 16, rgrp, jnp.int32(0))
        pltpu.sync_copy(ob_v, xts_hbm.at[cid, pl.ds(rb, HR)])


@functools.partial(
    pl.kernel,
    mesh=_mesh,
    compiler_params=_sc_params,
    out_type=jax.ShapeDtypeStruct((NC, NP, DH), f32),
    scratch_types=[
        [pltpu.VMEM((KE,), i32) for _ in range(4)],
        [pltpu.VMEM((KE,), i32) for _ in range(4)],
        [pltpu.VMEM((KE, DH), f32) for _ in range(2)],
        pltpu.VMEM((NZR, DH), f32),
        pltpu.VMEM((RPT,), f32),
        [pltpu.SemaphoreType.DMA for _ in range(4)],
        [pltpu.SemaphoreType.DMA for _ in range(2)],
        pltpu.VMEM_SHARED((NP, DH), f32),
    ],
)
def _hop_kernel(h_hbm, src_hbm, dst_hbm, sc_hbm, out_hbm,
                sidx, didx, rbuf, zero_v, sc_v, isems, gsems, acc_sp):
    cid = lax.axis_index("c")
    sid = lax.axis_index("s")

    def idx_load(c, q):
        row = sid * NCHH + c
        pltpu.async_copy(src_hbm.at[row], sidx[q], isems[q])
        pltpu.async_copy(dst_hbm.at[row], didx[q], isems[q])

    def idx_wait(c, q):
        row = sid * NCHH + c
        pltpu.make_async_copy(src_hbm.at[row], sidx[q], isems[q]).wait()
        pltpu.make_async_copy(dst_hbm.at[row], didx[q], isems[q]).wait()

    def gather_start(q, b):
        pltpu.async_copy(h_hbm.at[cid].at[sidx[q]], rbuf[b], gsems[b])

    def gather_wait(q, b):
        pltpu.make_async_copy(
            h_hbm.at[cid].at[sidx[q]], rbuf[b], gsems[b]).wait()

    zvec = jnp.zeros((16,), f32)
    for i in range(NZR):
        for j in range(DH // 16):
            zero_v[i, pl.ds(j * 16, 16)] = zvec
    for k in range(RPT // NZR):
        pltpu.sync_copy(zero_v, acc_sp.at[pl.ds(sid * RPT + k * NZR, NZR)])
    pltpu.sync_copy(sc_hbm.at[pl.ds(sid * RPT, RPT)], sc_v)
    plsc.subcore_barrier()

    for q in range(4):
        idx_load(q, q)
    idx_wait(0, 0)
    gather_start(0, 0)

    def quad(g, carry):
        for b4 in range(4):
            c = g * 4 + b4
            b = b4 % 2
            gather_wait(b4, b)
            c1 = c + 1

            @pl.when(c1 < NCHH)
            def _():
                idx_wait(c1, (b4 + 1) % 4)
                gather_start((b4 + 1) % 4, 1 - b)

            pltpu.sync_copy(rbuf[b], acc_sp.at[didx[b4]], add=True)
            c4 = c + 4

            @pl.when(c4 < NCHH)
            def _():
                idx_load(c4, b4)

        return carry

    lax.fori_loop(0, NCHH // 4, quad, jnp.int32(0))
    plsc.subcore_barrier()

    for k in range(2):
        rb = sid * RPT + k * HR
        pltpu.sync_copy(acc_sp.at[pl.ds(rb, HR)], rbuf[0].at[pl.ds(0, HR)])
        pltpu.sync_copy(h_hbm.at[cid, pl.ds(rb, HR)],
                        rbuf[1].at[pl.ds(0, HR)])

        def rgrp(g, carry):
            sv = sc_v[pl.ds(k * HR + g * 16, 16)]
            for rr in range(16):
                r = g * 16 + rr
                s = jnp.full((16,), sv[rr], f32)
                for j in range(DH // 16):
                    sl = pl.ds(j * 16, 16)
                    rbuf[0][r, sl] = (rbuf[0][r, sl] + rbuf[1][r, sl]) * s
            return carry

        lax.fori_loop(0, HR // 16, rgrp, jnp.int32(0))
        pltpu.sync_copy(rbuf[0].at[pl.ds(0, HR)],
                        out_hbm.at[cid, pl.ds(rb, HR)])


def _final_body(v_ref, w_ref, b_ref, out_ref):
    h = jnp.concatenate([v_ref[0], v_ref[1]], axis=1)
    out_ref[...] = lax.dot_general(
        h, w_ref[...], (((1,), (1,)), ((), ())),
        preferred_element_type=f32) + b_ref[...]


_final = pl.pallas_call(
    _final_body,
    grid=(NP // 1024,),
    in_specs=[
        pl.BlockSpec((NC, 1024, DH), lambda i: (0, i, 0)),
        pl.BlockSpec((D, D), lambda i: (0, 0)),
        pl.BlockSpec((1, D), lambda i: (0, 0)),
    ],
    out_specs=pl.BlockSpec((1024, D), lambda i: (i, 0)),
    out_shape=jax.ShapeDtypeStruct((NP, D), f32),
)


def kernel(x, edge_index, W, b):
    ei = edge_index.astype(i32)
    pad = N + (jnp.arange(E_PAD - E, dtype=i32) % (NP - N))
    src = jnp.concatenate([ei[0], pad]).reshape(E_PAD // KE, KE)
    dst = jnp.concatenate([ei[1], pad]).reshape(E_PAD // KE, KE)
    x_p = jnp.pad(x, ((0, NP - N), (0, 0)))

    xts, dis, dinv = _prep_kernel(dst, x_p)
    h1s = _hop_kernel(xts, src, dst, dinv)
    vs = _hop_kernel(h1s, src, dst, dis)
    out = _final(vs, W, b.reshape(1, D))
    return out[:N]


# speedup vs baseline: 30.8044x; 1.0127x over previous
import functools

import jax
import jax.numpy as jnp
from jax import lax
from jax.experimental import pallas as pl
from jax.experimental.pallas import tpu as pltpu
from jax.experimental.pallas import tpu_sc as plsc

N = 10000
NP = 10240
E = 320000
D = 128
DH = D // 2
NC, NS = 2, 16
KE = 512
E_PAD = 327680
NCHH = E_PAD // (NS * KE)
RPT = NP // NS
HR = RPT // 2
NZR = 16

f32 = jnp.float32
i32 = jnp.int32

_mesh = plsc.VectorSubcoreMesh(core_axis_name="c", subcore_axis_name="s")
_sc_params = pltpu.CompilerParams(use_tc_tiling_on_sc=False)


def _rsqrt_newton(d):
    y = lax.bitcast_convert_type(
        jnp.int32(0x5F3759DF) - (lax.bitcast_convert_type(d, i32) >> 1), f32)
    for _ in range(3):
        y = y * (1.5 - 0.5 * d * y * y)
    return y


@functools.partial(
    pl.kernel,
    mesh=_mesh,
    compiler_params=_sc_params,
    out_type=(
        jax.ShapeDtypeStruct((NC, NP, DH), f32),
        jax.ShapeDtypeStruct((NP,), f32),
        jax.ShapeDtypeStruct((NP,), f32),
    ),
    scratch_types=[
        pltpu.VMEM((NCHH, KE), i32),
        pltpu.VMEM((KE,), f32),
        pltpu.VMEM((RPT,), f32),
        pltpu.VMEM((RPT,), f32),
        pltpu.VMEM((RPT,), f32),
        pltpu.VMEM((HR, D), f32),
        pltpu.VMEM((HR, DH), f32),
        pltpu.SemaphoreType.DMA,
        pltpu.VMEM_SHARED((NP,), f32),
    ],
)
def _prep_kernel(dst_hbm, x_hbm, xts_hbm, dis_hbm, dinv_hbm,
                 dall_v, ones_v, deg_v, dis_v, dinv_v, xb_v, ob_v,
                 ssem, deg_sp):
    cid = lax.axis_index("c")
    sid = lax.axis_index("s")

    def fill(ref, n, val):
        v = jnp.full((16,), val, f32)
        for i in range(n // 16):
            ref[pl.ds(i * 16, 16)] = v

    fill(ones_v, KE, 1.0)
    fill(deg_v, RPT, 0.0)
    pltpu.sync_copy(deg_v, deg_sp.at[pl.ds(sid * RPT, RPT)])
    pltpu.sync_copy(dst_hbm.at[pl.ds(sid * NCHH, NCHH)], dall_v)
    plsc.subcore_barrier()

    def body(c, carry):
        pltpu.sync_copy(ones_v, deg_sp.at[dall_v.at[c]], add=True)
        return carry

    lax.fori_loop(0, NCHH, body, jnp.int32(0))
    plsc.subcore_barrier()

    base = sid * RPT
    pltpu.sync_copy(deg_sp.at[pl.ds(base, RPT)], deg_v)

    def grp(i, carry):
        d = deg_v[pl.ds(i * 16, 16)] + 1.0
        y = _rsqrt_newton(d)
        dis_v[pl.ds(i * 16, 16)] = y
        dinv_v[pl.ds(i * 16, 16)] = 1.0 / d
        return carry

    lax.fori_loop(0, RPT // 16, grp, jnp.int32(0))

    @pl.when(cid == 0)
    def _():
        pltpu.sync_copy(dis_v, dis_hbm.at[pl.ds(base, RPT)])
        pltpu.sync_copy(dinv_v, dinv_hbm.at[pl.ds(base, RPT)])

    for k in range(2):
        rb = base + k * HR
        pltpu.sync_copy(x_hbm.at[pl.ds(rb, HR)], xb_v)

        def rgrp(g, carry):
            sv = dis_v[pl.ds(k * HR + g * 16, 16)]
            for rr in range(16):
                r = g * 16 + rr
                s = jnp.full((16,), sv[rr], f32)
                for j in range(DH // 16):
                    ob_v[r, pl.ds(j * 16, 16)] = (
                        xb_v[r, pl.ds(cid * DH + j * 16, 16)] * s)
            return carry

        lax.fori_loop(0, HR // 16, rgrp, jnp.int32(0))
        pltpu.sync_copy(ob_v, xts_hbm.at[cid, pl.ds(rb, HR)])


@functools.partial(
    pl.kernel,
    mesh=_mesh,
    compiler_params=_sc_params,
    out_type=jax.ShapeDtypeStruct((NC, NP, DH), f32),
    scratch_types=[
        [pltpu.VMEM((KE,), i32) for _ in range(4)],
        [pltpu.VMEM((KE,), i32) for _ in range(4)],
        [pltpu.VMEM((KE, DH), f32) for _ in range(2)],
        pltpu.VMEM((NZR, DH), f32),
        pltpu.VMEM((RPT,), f32),
        [pltpu.SemaphoreType.DMA for _ in range(4)],
        [pltpu.SemaphoreType.DMA for _ in range(2)],
        pltpu.VMEM_SHARED((NP, DH), f32),
    ],
)
def _hop_kernel(h_hbm, src_hbm, dst_hbm, sc_hbm, out_hbm,
                sidx, didx, rbuf, zero_v, sc_v, isems, gsems, acc_sp):
    cid = lax.axis_index("c")
    sid = lax.axis_index("s")

    def idx_load(c, q):
        row = sid * NCHH + c
        pltpu.async_copy(src_hbm.at[row], sidx[q], isems[q])
        pltpu.async_copy(dst_hbm.at[row], didx[q], isems[q])

    def idx_wait(c, q):
        row = sid * NCHH + c
        pltpu.make_async_copy(src_hbm.at[row], sidx[q], isems[q]).wait()
        pltpu.make_async_copy(dst_hbm.at[row], didx[q], isems[q]).wait()

    def gather_start(q, b):
        pltpu.async_copy(h_hbm.at[cid].at[sidx[q]], rbuf[b], gsems[b])

    def gather_wait(q, b):
        pltpu.make_async_copy(
            h_hbm.at[cid].at[sidx[q]], rbuf[b], gsems[b]).wait()

    zvec = jnp.zeros((16,), f32)
    for i in range(NZR):
        for j in range(DH // 16):
            zero_v[i, pl.ds(j * 16, 16)] = zvec
    for k in range(RPT // NZR):
        pltpu.sync_copy(zero_v, acc_sp.at[pl.ds(sid * RPT + k * NZR, NZR)])
    pltpu.sync_copy(sc_hbm.at[pl.ds(sid * RPT, RPT)], sc_v)
    plsc.subcore_barrier()

    for q in range(4):
        idx_load(q, q)
    idx_wait(0, 0)
    gather_start(0, 0)

    def quad(g, carry):
        for b4 in range(4):
            c = g * 4 + b4
            b = b4 % 2
            gather_wait(b4, b)
            c1 = c + 1

            @pl.when(c1 < NCHH)
            def _():
                idx_wait(c1, (b4 + 1) % 4)
                gather_start((b4 + 1) % 4, 1 - b)

            pltpu.sync_copy(rbuf[b], acc_sp.at[didx[b4]], add=True)
            c4 = c + 4

            @pl.when(c4 < NCHH)
            def _():
                idx_load(c4, b4)

        return carry

    lax.fori_loop(0, NCHH // 4, quad, jnp.int32(0))
    plsc.subcore_barrier()

    for k in range(2):
        rb = sid * RPT + k * HR
        pltpu.sync_copy(acc_sp.at[pl.ds(rb, HR)], rbuf[0].at[pl.ds(0, HR)])
        pltpu.sync_copy(h_hbm.at[cid, pl.ds(rb, HR)],
                        rbuf[1].at[pl.ds(0, HR)])

        def rgrp(g, carry):
            sv = sc_v[pl.ds(k * HR + g * 16, 16)]
            for rr in range(16):
                r = g * 16 + rr
                s = jnp.full((16,), sv[rr], f32)
                for j in range(DH // 16):
                    sl = pl.ds(j * 16, 16)
                    rbuf[0][r, sl] = (rbuf[0][r, sl] + rbuf[1][r, sl]) * s
            return carry

        lax.fori_loop(0, HR // 16, rgrp, jnp.int32(0))
        pltpu.sync_copy(rbuf[0].at[pl.ds(0, HR)],
                        out_hbm.at[cid, pl.ds(rb, HR)])


def _final_body(v_ref, w_ref, b_ref, out_ref):
    h = jnp.concatenate([v_ref[0], v_ref[1]], axis=1)
    out_ref[...] = lax.dot_general(
        h, w_ref[...], (((1,), (1,)), ((), ())),
        preferred_element_type=f32) + b_ref[...]


_final = pl.pallas_call(
    _final_body,
    grid=(NP // 1024,),
    in_specs=[
        pl.BlockSpec((NC, 1024, DH), lambda i: (0, i, 0)),
        pl.BlockSpec((D, D), lambda i: (0, 0)),
        pl.BlockSpec((1, D), lambda i: (0, 0)),
    ],
    out_specs=pl.BlockSpec((1024, D), lambda i: (i, 0)),
    out_shape=jax.ShapeDtypeStruct((N, D), f32),
)


def kernel(x, edge_index, W, b):
    ei = edge_index.astype(i32)
    pad = N + (jnp.arange(E_PAD - E, dtype=i32) % (NP - N))
    src = jnp.concatenate([ei[0], pad]).reshape(E_PAD // KE, KE)
    dst = jnp.concatenate([ei[1], pad]).reshape(E_PAD // KE, KE)
    x_p = jnp.pad(x, ((0, NP - N), (0, 0)))

    xts, dis, dinv = _prep_kernel(dst, x_p)
    h1s = _hop_kernel(xts, src, dst, dinv)
    vs = _hop_kernel(h1s, src, dst, dis)
    return _final(vs, W, b.reshape(1, D))


# speedup vs baseline: 31.0763x; 1.0088x over previous
import functools

import jax
import jax.numpy as jnp
from jax import lax
from jax.experimental import pallas as pl
from jax.experimental.pallas import tpu as pltpu
from jax.experimental.pallas import tpu_sc as plsc

N = 10000
NP = 10240
E = 320000
D = 128
DH = D // 2
NC, NS = 2, 16
KE = 512
E_PAD = 327680
NCHH = E_PAD // (NS * KE)
RPT = NP // NS
HR = RPT // 2
NZR = 16

f32 = jnp.float32
i32 = jnp.int32

_mesh = plsc.VectorSubcoreMesh(core_axis_name="c", subcore_axis_name="s")
_sc_params = pltpu.CompilerParams(use_tc_tiling_on_sc=False)


def _rsqrt_newton(d):
    y = lax.bitcast_convert_type(
        jnp.int32(0x5F3759DF) - (lax.bitcast_convert_type(d, i32) >> 1), f32)
    for _ in range(3):
        y = y * (1.5 - 0.5 * d * y * y)
    return y


@functools.partial(
    pl.kernel,
    mesh=_mesh,
    compiler_params=_sc_params,
    out_type=(
        jax.ShapeDtypeStruct((NC, NP, DH), f32),
        jax.ShapeDtypeStruct((NP,), f32),
        jax.ShapeDtypeStruct((NP,), f32),
    ),
    scratch_types=[
        pltpu.VMEM((NCHH, KE), i32),
        pltpu.VMEM((KE,), f32),
        pltpu.VMEM((RPT,), f32),
        pltpu.VMEM((RPT,), f32),
        pltpu.VMEM((RPT,), f32),
        pltpu.VMEM((HR, D), f32),
        pltpu.VMEM((HR, DH), f32),
        pltpu.SemaphoreType.DMA,
        pltpu.VMEM_SHARED((NP,), f32),
    ],
)
def _prep_kernel(dst_hbm, x_hbm, xts_hbm, dis_hbm, dinv_hbm,
                 dall_v, ones_v, deg_v, dis_v, dinv_v, xb_v, ob_v,
                 ssem, deg_sp):
    cid = lax.axis_index("c")
    sid = lax.axis_index("s")

    def fill(ref, n, val):
        v = jnp.full((16,), val, f32)
        for i in range(n // 16):
            ref[pl.ds(i * 16, 16)] = v

    fill(ones_v, KE, 1.0)
    fill(deg_v, RPT, 0.0)
    pltpu.sync_copy(deg_v, deg_sp.at[pl.ds(sid * RPT, RPT)])
    pltpu.sync_copy(dst_hbm.at[pl.ds(sid * NCHH, NCHH)], dall_v)
    pltpu.async_copy(x_hbm.at[pl.ds(sid * RPT, HR)], xb_v, ssem)
    plsc.subcore_barrier()

    def body(c, carry):
        pltpu.sync_copy(ones_v, deg_sp.at[dall_v.at[c]], add=True)
        return carry

    lax.fori_loop(0, NCHH, body, jnp.int32(0))
    plsc.subcore_barrier()

    base = sid * RPT
    pltpu.sync_copy(deg_sp.at[pl.ds(base, RPT)], deg_v)

    def grp(i, carry):
        d = deg_v[pl.ds(i * 16, 16)] + 1.0
        y = _rsqrt_newton(d)
        dis_v[pl.ds(i * 16, 16)] = y
        dinv_v[pl.ds(i * 16, 16)] = 1.0 / d
        return carry

    lax.fori_loop(0, RPT // 16, grp, jnp.int32(0))

    @pl.when(cid == 0)
    def _():
        pltpu.sync_copy(dis_v, dis_hbm.at[pl.ds(base, RPT)])
        pltpu.sync_copy(dinv_v, dinv_hbm.at[pl.ds(base, RPT)])

    for k in range(2):
        rb = base + k * HR
        if k == 0:
            pltpu.make_async_copy(
                x_hbm.at[pl.ds(sid * RPT, HR)], xb_v, ssem).wait()
        else:
            pltpu.sync_copy(x_hbm.at[pl.ds(rb, HR)], xb_v)

        def rgrp(g, carry):
            sv = dis_v[pl.ds(k * HR + g * 16, 16)]
            for rr in range(16):
                r = g * 16 + rr
                s = jnp.full((16,), sv[rr], f32)
                for j in range(DH // 16):
                    ob_v[r, pl.ds(j * 16, 16)] = (
                        xb_v[r, pl.ds(cid * DH + j * 16, 16)] * s)
            return carry

        lax.fori_loop(0, HR // 16, rgrp, jnp.int32(0))
        pltpu.sync_copy(ob_v, xts_hbm.at[cid, pl.ds(rb, HR)])


@functools.partial(
    pl.kernel,
    mesh=_mesh,
    compiler_params=_sc_params,
    out_type=jax.ShapeDtypeStruct((NC, NP, DH), f32),
    scratch_types=[
        [pltpu.VMEM((KE,), i32) for _ in range(4)],
        [pltpu.VMEM((KE,), i32) for _ in range(4)],
        [pltpu.VMEM((KE, DH), f32) for _ in range(2)],
        pltpu.VMEM((NZR, DH), f32),
        pltpu.VMEM((RPT,), f32),
        [pltpu.SemaphoreType.DMA for _ in range(4)],
        [pltpu.SemaphoreType.DMA for _ in range(2)],
        pltpu.VMEM_SHARED((NP, DH), f32),
    ],
)
def _hop_kernel(h_hbm, src_hbm, dst_hbm, sc_hbm, out_hbm,
                sidx, didx, rbuf, zero_v, sc_v, isems, gsems, acc_sp):
    cid = lax.axis_index("c")
    sid = lax.axis_index("s")

    def idx_load(c, q):
        row = sid * NCHH + c
        pltpu.async_copy(src_hbm.at[row], sidx[q], isems[q])
        pltpu.async_copy(dst_hbm.at[row], didx[q], isems[q])

    def idx_wait(c, q):
        row = sid * NCHH + c
        pltpu.make_async_copy(src_hbm.at[row], sidx[q], isems[q]).wait()
        pltpu.make_async_copy(dst_hbm.at[row], didx[q], isems[q]).wait()

    def gather_start(q, b):
        pltpu.async_copy(h_hbm.at[cid].at[sidx[q]], rbuf[b], gsems[b])

    def gather_wait(q, b):
        pltpu.make_async_copy(
            h_hbm.at[cid].at[sidx[q]], rbuf[b], gsems[b]).wait()

    zvec = jnp.zeros((16,), f32)
    for i in range(NZR):
        for j in range(DH // 16):
            zero_v[i, pl.ds(j * 16, 16)] = zvec
    for k in range(RPT // NZR):
        pltpu.sync_copy(zero_v, acc_sp.at[pl.ds(sid * RPT + k * NZR, NZR)])
    pltpu.sync_copy(sc_hbm.at[pl.ds(sid * RPT, RPT)], sc_v)

    for q in range(4):
        idx_load(q, q)
    idx_wait(0, 0)
    gather_start(0, 0)
    plsc.subcore_barrier()

    def quad(g, carry):
        for b4 in range(4):
            c = g * 4 + b4
            b = b4 % 2
            gather_wait(b4, b)
            c1 = c + 1

            @pl.when(c1 < NCHH)
            def _():
                idx_wait(c1, (b4 + 1) % 4)
                gather_start((b4 + 1) % 4, 1 - b)

            pltpu.sync_copy(rbuf[b], acc_sp.at[didx[b4]], add=True)
            c4 = c + 4

            @pl.when(c4 < NCHH)
            def _():
                idx_load(c4, b4)

        return carry

    lax.fori_loop(0, NCHH // 4, quad, jnp.int32(0))
    plsc.subcore_barrier()

    for k in range(2):
        rb = sid * RPT + k * HR
        pltpu.sync_copy(acc_sp.at[pl.ds(rb, HR)], rbuf[0].at[pl.ds(0, HR)])
        pltpu.sync_copy(h_hbm.at[cid, pl.ds(rb, HR)],
                        rbuf[1].at[pl.ds(0, HR)])

        def rgrp(g, carry):
            sv = sc_v[pl.ds(k * HR + g * 16, 16)]
            for rr in range(16):
                r = g * 16 + rr
                s = jnp.full((16,), sv[rr], f32)
                for j in range(DH // 16):
                    sl = pl.ds(j * 16, 16)
                    rbuf[0][r, sl] = (rbuf[0][r, sl] + rbuf[1][r, sl]) * s
            return carry

        lax.fori_loop(0, HR // 16, rgrp, jnp.int32(0))
        pltpu.sync_copy(rbuf[0].at[pl.ds(0, HR)],
                        out_hbm.at[cid, pl.ds(rb, HR)])


def _final_body(v_ref, w_ref, b_ref, out_ref):
    h = jnp.concatenate([v_ref[0], v_ref[1]], axis=1)
    out_ref[...] = lax.dot_general(
        h, w_ref[...], (((1,), (1,)), ((), ())),
        preferred_element_type=f32) + b_ref[...]


_final = pl.pallas_call(
    _final_body,
    grid=(NP // 1024,),
    in_specs=[
        pl.BlockSpec((NC, 1024, DH), lambda i: (0, i, 0)),
        pl.BlockSpec((D, D), lambda i: (0, 0)),
        pl.BlockSpec((1, D), lambda i: (0, 0)),
    ],
    out_specs=pl.BlockSpec((1024, D), lambda i: (i, 0)),
    out_shape=jax.ShapeDtypeStruct((N, D), f32),
)


def kernel(x, edge_index, W, b):
    ei = edge_index.astype(i32)
    pad = N + (jnp.arange(E_PAD - E, dtype=i32) % (NP - N))
    src = jnp.concatenate([ei[0], pad]).reshape(E_PAD // KE, KE)
    dst = jnp.concatenate([ei[1], pad]).reshape(E_PAD // KE, KE)
    x_p = jnp.pad(x, ((0, NP - N), (0, 0)))

    xts, dis, dinv = _prep_kernel(dst, x_p)
    h1s = _hop_kernel(xts, src, dst, dinv)
    vs = _hop_kernel(h1s, src, dst, dis)
    return _final(vs, W, b.reshape(1, D))


# speedup vs baseline: 31.9994x; 1.0297x over previous
import functools

import jax
import jax.numpy as jnp
from jax import lax
from jax.experimental import pallas as pl
from jax.experimental.pallas import tpu as pltpu
from jax.experimental.pallas import tpu_sc as plsc

N = 10000
NP = 10240
E = 320000
D = 128
DH = D // 2
NC, NS = 2, 16
KE = 512
E_PAD = 327680
NCHH = E_PAD // (NS * KE)
NCHR = E // KE
NCHP = (E_PAD - E) // KE
RPT = NP // NS
HR = RPT // 2
NZR = 16

f32 = jnp.float32
i32 = jnp.int32

_mesh = plsc.VectorSubcoreMesh(core_axis_name="c", subcore_axis_name="s")
_sc_params = pltpu.CompilerParams(use_tc_tiling_on_sc=False)


def _rsqrt_newton(d):
    y = lax.bitcast_convert_type(
        jnp.int32(0x5F3759DF) - (lax.bitcast_convert_type(d, i32) >> 1), f32)
    for _ in range(3):
        y = y * (1.5 - 0.5 * d * y * y)
    return y


@functools.partial(
    pl.kernel,
    mesh=_mesh,
    compiler_params=_sc_params,
    out_type=(
        jax.ShapeDtypeStruct((NC, NP, DH), f32),
        jax.ShapeDtypeStruct((NP,), f32),
        jax.ShapeDtypeStruct((NP,), f32),
    ),
    scratch_types=[
        pltpu.VMEM((NCHH, KE), i32),
        pltpu.VMEM((KE,), f32),
        pltpu.VMEM((RPT,), f32),
        pltpu.VMEM((RPT,), f32),
        pltpu.VMEM((RPT,), f32),
        pltpu.VMEM((HR, D), f32),
        pltpu.VMEM((HR, DH), f32),
        pltpu.SemaphoreType.DMA,
        pltpu.VMEM_SHARED((NP,), f32),
    ],
)
def _prep_kernel(ei_hbm, pad_hbm, x_hbm, xts_hbm, dis_hbm, dinv_hbm,
                 dall_v, ones_v, deg_v, dis_v, dinv_v, xb_v, ob_v,
                 ssem, deg_sp):
    cid = lax.axis_index("c")
    sid = lax.axis_index("s")

    def fill(ref, n, val):
        v = jnp.full((16,), val, f32)
        for i in range(n // 16):
            ref[pl.ds(i * 16, 16)] = v

    fill(ones_v, KE, 1.0)
    fill(deg_v, RPT, 0.0)
    pltpu.sync_copy(deg_v, deg_sp.at[pl.ds(sid * RPT, RPT)])

    @pl.when(sid < NS - 1)
    def _():
        pltpu.sync_copy(ei_hbm.at[1].at[pl.ds(sid * NCHH, NCHH)], dall_v)

    @pl.when(sid == NS - 1)
    def _():
        nr = NCHR - (NS - 1) * NCHH
        pltpu.sync_copy(ei_hbm.at[1].at[pl.ds((NS - 1) * NCHH, nr)],
                        dall_v.at[pl.ds(0, nr)])
        pltpu.sync_copy(pad_hbm.at[1], dall_v.at[pl.ds(nr, NCHP)])
    pltpu.async_copy(x_hbm.at[pl.ds(sid * RPT, HR)], xb_v, ssem)
    plsc.subcore_barrier()

    def body(c, carry):
        pltpu.sync_copy(ones_v, deg_sp.at[dall_v.at[c]], add=True)
        return carry

    lax.fori_loop(0, NCHH, body, jnp.int32(0))
    plsc.subcore_barrier()

    base = sid * RPT
    pltpu.sync_copy(deg_sp.at[pl.ds(base, RPT)], deg_v)

    def grp(i, carry):
        d = deg_v[pl.ds(i * 16, 16)] + 1.0
        y = _rsqrt_newton(d)
        dis_v[pl.ds(i * 16, 16)] = y
        dinv_v[pl.ds(i * 16, 16)] = 1.0 / d
        return carry

    lax.fori_loop(0, RPT // 16, grp, jnp.int32(0))

    @pl.when(cid == 0)
    def _():
        pltpu.sync_copy(dis_v, dis_hbm.at[pl.ds(base, RPT)])
        pltpu.sync_copy(dinv_v, dinv_hbm.at[pl.ds(base, RPT)])

    for k in range(2):
        rb = base + k * HR
        if k == 0:
            pltpu.make_async_copy(
                x_hbm.at[pl.ds(sid * RPT, HR)], xb_v, ssem).wait()
        else:
            pltpu.sync_copy(x_hbm.at[pl.ds(rb, HR)], xb_v)

        def rgrp(g, carry):
            sv = dis_v[pl.ds(k * HR + g * 16, 16)]
            for rr in range(16):
                r = g * 16 + rr
                s = jnp.full((16,), sv[rr], f32)
                for j in range(DH // 16):
                    ob_v[r, pl.ds(j * 16, 16)] = (
                        xb_v[r, pl.ds(cid * DH + j * 16, 16)] * s)
            return carry

        lax.fori_loop(0, HR // 16, rgrp, jnp.int32(0))
        pltpu.sync_copy(ob_v, xts_hbm.at[cid, pl.ds(rb, HR)])


@functools.partial(
    pl.kernel,
    mesh=_mesh,
    compiler_params=_sc_params,
    out_type=jax.ShapeDtypeStruct((NC, NP, DH), f32),
    scratch_types=[
        [pltpu.VMEM((KE,), i32) for _ in range(4)],
        [pltpu.VMEM((KE,), i32) for _ in range(4)],
        [pltpu.VMEM((KE, DH), f32) for _ in range(2)],
        pltpu.VMEM((NZR, DH), f32),
        pltpu.VMEM((RPT,), f32),
        [pltpu.SemaphoreType.DMA for _ in range(4)],
        [pltpu.SemaphoreType.DMA for _ in range(2)],
        pltpu.VMEM_SHARED((NP, DH), f32),
    ],
)
def _hop_kernel(h_hbm, ei_hbm, pad_hbm, sc_hbm, out_hbm,
                sidx, didx, rbuf, zero_v, sc_v, isems, gsems, acc_sp):
    cid = lax.axis_index("c")
    sid = lax.axis_index("s")

    def idx_load(c, q):
        row = sid * NCHH + c

        @pl.when(row < NCHR)
        def _():
            pltpu.async_copy(ei_hbm.at[0].at[row], sidx[q], isems[q])
            pltpu.async_copy(ei_hbm.at[1].at[row], didx[q], isems[q])

        @pl.when(row >= NCHR)
        def _():
            pltpu.async_copy(pad_hbm.at[0].at[row - NCHR], sidx[q], isems[q])
            pltpu.async_copy(pad_hbm.at[1].at[row - NCHR], didx[q], isems[q])

    def idx_wait(c, q):
        pltpu.make_async_copy(ei_hbm.at[0].at[0], sidx[q], isems[q]).wait()
        pltpu.make_async_copy(ei_hbm.at[1].at[0], didx[q], isems[q]).wait()

    def gather_start(q, b):
        pltpu.async_copy(h_hbm.at[cid].at[sidx[q]], rbuf[b], gsems[b])

    def gather_wait(q, b):
        pltpu.make_async_copy(
            h_hbm.at[cid].at[sidx[q]], rbuf[b], gsems[b]).wait()

    zvec = jnp.zeros((16,), f32)
    for i in range(NZR):
        for j in range(DH // 16):
            zero_v[i, pl.ds(j * 16, 16)] = zvec
    for k in range(RPT // NZR):
        pltpu.sync_copy(zero_v, acc_sp.at[pl.ds(sid * RPT + k * NZR, NZR)])
    pltpu.sync_copy(sc_hbm.at[pl.ds(sid * RPT, RPT)], sc_v)

    for q in range(4):
        idx_load(q, q)
    idx_wait(0, 0)
    gather_start(0, 0)
    plsc.subcore_barrier()

    def quad(g, carry):
        for b4 in range(4):
            c = g * 4 + b4
            b = b4 % 2
            gather_wait(b4, b)
            c1 = c + 1

            @pl.when(c1 < NCHH)
            def _():
                idx_wait(c1, (b4 + 1) % 4)
                gather_start((b4 + 1) % 4, 1 - b)

            pltpu.sync_copy(rbuf[b], acc_sp.at[didx[b4]], add=True)
            c4 = c + 4

            @pl.when(c4 < NCHH)
            def _():
                idx_load(c4, b4)

        return carry

    lax.fori_loop(0, NCHH // 4, quad, jnp.int32(0))
    plsc.subcore_barrier()

    for k in range(2):
        rb = sid * RPT + k * HR
        pltpu.sync_copy(acc_sp.at[pl.ds(rb, HR)], rbuf[0].at[pl.ds(0, HR)])
        pltpu.sync_copy(h_hbm.at[cid, pl.ds(rb, HR)],
                        rbuf[1].at[pl.ds(0, HR)])

        def rgrp(g, carry):
            sv = sc_v[pl.ds(k * HR + g * 16, 16)]
            for rr in range(16):
                r = g * 16 + rr
                s = jnp.full((16,), sv[rr], f32)
                for j in range(DH // 16):
                    sl = pl.ds(j * 16, 16)
                    rbuf[0][r, sl] = (rbuf[0][r, sl] + rbuf[1][r, sl]) * s
            return carry

        lax.fori_loop(0, HR // 16, rgrp, jnp.int32(0))
        pltpu.sync_copy(rbuf[0].at[pl.ds(0, HR)],
                        out_hbm.at[cid, pl.ds(rb, HR)])


def _final_body(v_ref, w_ref, b_ref, out_ref):
    h = jnp.concatenate([v_ref[0], v_ref[1]], axis=1)
    out_ref[...] = lax.dot_general(
        h, w_ref[...], (((1,), (1,)), ((), ())),
        preferred_element_type=f32) + b_ref[...]


_final = pl.pallas_call(
    _final_body,
    grid=(NP // 1024,),
    in_specs=[
        pl.BlockSpec((NC, 1024, DH), lambda i: (0, i, 0)),
        pl.BlockSpec((D, D), lambda i: (0, 0)),
        pl.BlockSpec((1, D), lambda i: (0, 0)),
    ],
    out_specs=pl.BlockSpec((1024, D), lambda i: (i, 0)),
    out_shape=jax.ShapeDtypeStruct((N, D), f32),
)


def kernel(x, edge_index, W, b):
    ei3 = edge_index.astype(i32).reshape(2, NCHR, KE)
    padr = (N + (jnp.arange(E_PAD - E, dtype=i32) % (NP - N))).reshape(
        NCHP, KE)
    pad3 = jnp.stack([padr, padr])
    x_p = jnp.pad(x, ((0, NP - N), (0, 0)))

    xts, dis, dinv = _prep_kernel(ei3, pad3, x_p)
    h1s = _hop_kernel(xts, ei3, pad3, dinv)
    vs = _hop_kernel(h1s, ei3, pad3, dis)
    return _final(vs, W, b.reshape(1, D))
